# baseline (speedup 1.0000x reference)
import functools

import jax
import jax.numpy as jnp
from jax import lax
from jax.experimental import pallas as pl
from jax.experimental.pallas import tpu as pltpu
from jax.experimental.pallas import tpu_sc as plsc

EMB = 32
HIDDEN = 128
B = 16384
PACK = 128 // EMB

NC = 2
NS = 16
NW = NC * NS
B_PER_W = B // NW
CHUNK = 256
N_CHUNKS = B_PER_W // CHUNK


def _make_gather():
    mesh = plsc.VectorSubcoreMesh(core_axis_name="c", subcore_axis_name="s")

    @functools.partial(
        pl.kernel,
        out_type=[
            jax.ShapeDtypeStruct((B, 128), jnp.float32),
            jax.ShapeDtypeStruct((B, 128), jnp.float32),
        ],
        mesh=mesh,
        scratch_types=[
            pltpu.VMEM((B_PER_W,), jnp.int32),
            pltpu.VMEM((B_PER_W,), jnp.int32),
            pltpu.VMEM((CHUNK,), jnp.int32),
            pltpu.VMEM((CHUNK,), jnp.int32),
            pltpu.VMEM((CHUNK, 128), jnp.float32),
            pltpu.VMEM((CHUNK, 128), jnp.float32),
            pltpu.SemaphoreType.DMA,
            pltpu.SemaphoreType.DMA,
        ],
    )
    def gather_k(uid_hbm, mid_hbm, uemb_hbm, memb_hbm, uout_hbm, mout_hbm,
                 uidx_v, midx_v, urow_v, mrow_v, urows_v, mrows_v, usem, msem):
        wid = lax.axis_index("s") * NC + lax.axis_index("c")
        base = wid * B_PER_W
        pltpu.sync_copy(uid_hbm.at[pl.ds(base, B_PER_W)], uidx_v)
        pltpu.sync_copy(mid_hbm.at[pl.ds(base, B_PER_W)], midx_v)

        def chunk_body(c, _):
            cb = c * CHUNK
            for v in range(CHUNK // 16):
                sl = pl.ds(cb + v * 16, 16)
                dl = pl.ds(v * 16, 16)
                urow_v[dl] = lax.shift_right_logical(uidx_v[sl], 2)
                mrow_v[dl] = lax.shift_right_logical(midx_v[sl], 2)
            cu = pltpu.async_copy(uemb_hbm.at[urow_v], urows_v, usem)
            cm = pltpu.async_copy(memb_hbm.at[mrow_v], mrows_v, msem)
            cu.wait()
            cm.wait()
            pltpu.sync_copy(urows_v, uout_hbm.at[pl.ds(base + cb, CHUNK)])
            pltpu.sync_copy(mrows_v, mout_hbm.at[pl.ds(base + cb, CHUNK)])
            return ()

        lax.fori_loop(0, N_CHUNKS, chunk_body, (), unroll=True)

    return gather_k


_gather = _make_gather()

_BLK = 2048


def _mlp_body(u_ref, m_ref, uo_ref, mo_ref, w1u_ref, w1m_ref, b1_ref,
              w2_ref, b2_ref, o_ref):
    uoff = uo_ref[...]
    moff = mo_ref[...]
    u128 = u_ref[...]
    m128 = m_ref[...]
    usel = jnp.zeros((_BLK, EMB), jnp.float32)
    msel = jnp.zeros((_BLK, EMB), jnp.float32)
    for k in range(PACK):
        uk = (uoff == k).astype(jnp.float32)
        mk = (moff == k).astype(jnp.float32)
        usel = usel + uk * u128[:, k * EMB:(k + 1) * EMB]
        msel = msel + mk * m128[:, k * EMB:(k + 1) * EMB]
    h = (jnp.dot(usel, w1u_ref[...], preferred_element_type=jnp.float32)
         + jnp.dot(msel, w1m_ref[...], preferred_element_type=jnp.float32)
         + b1_ref[...])
    h = jnp.maximum(h, 0.0)
    o_ref[...] = (jnp.dot(h, w2_ref[...], preferred_element_type=jnp.float32)
                  + b2_ref[...])


def _mlp(u_rows, m_rows, uoff, moff, w1u, w1m, b1, w2, b2):
    return pl.pallas_call(
        _mlp_body,
        grid=(B // _BLK,),
        in_specs=[
            pl.BlockSpec((_BLK, 128), lambda i: (i, 0)),
            pl.BlockSpec((_BLK, 128), lambda i: (i, 0)),
            pl.BlockSpec((_BLK, 1), lambda i: (i, 0)),
            pl.BlockSpec((_BLK, 1), lambda i: (i, 0)),
            pl.BlockSpec((EMB, HIDDEN), lambda i: (0, 0)),
            pl.BlockSpec((EMB, HIDDEN), lambda i: (0, 0)),
            pl.BlockSpec((1, HIDDEN), lambda i: (0, 0)),
            pl.BlockSpec((HIDDEN, 1), lambda i: (0, 0)),
            pl.BlockSpec((1, 1), lambda i: (0, 0)),
        ],
        out_specs=pl.BlockSpec((_BLK, 1), lambda i: (i, 0)),
        out_shape=jax.ShapeDtypeStruct((B, 1), jnp.float32),
    )(u_rows, m_rows, uoff, moff, w1u, w1m, b1, w2, b2)


def kernel(user_ids, movie_ids, user_emb, movie_emb, W1, b1, W2, b2):
    uid = user_ids.astype(jnp.int32)
    mid = movie_ids.astype(jnp.int32)
    n_users = user_emb.shape[0]
    n_movies = movie_emb.shape[0]
    u128 = user_emb.reshape(n_users // PACK, 128)
    m128 = movie_emb.reshape(n_movies // PACK, 128)
    u_rows, m_rows = _gather(uid, mid, u128, m128)
    y = _mlp(u_rows, m_rows, (uid % PACK).reshape(B, 1),
             (mid % PACK).reshape(B, 1), W1[:EMB], W1[EMB:],
             b1.reshape(1, HIDDEN), W2, b2.reshape(1, 1))
    return y.reshape(-1)


# speedup vs baseline: 1.3803x; 1.3803x over previous
import functools

import jax
import jax.numpy as jnp
from jax import lax
from jax.experimental import pallas as pl
from jax.experimental.pallas import tpu as pltpu
from jax.experimental.pallas import tpu_sc as plsc

EMB = 32
HIDDEN = 128
B = 16384

NC = 2
NS = 16
NW = NC * NS
B_PER_W = B // NW
CH = 32
N_CHUNKS = B_PER_W // CH


def _make_gather():
    mesh = plsc.VectorSubcoreMesh(core_axis_name="c", subcore_axis_name="s")

    @functools.partial(
        pl.kernel,
        out_type=[
            jax.ShapeDtypeStruct((EMB, B), jnp.float32),
            jax.ShapeDtypeStruct((EMB, B), jnp.float32),
        ],
        mesh=mesh,
        scratch_types=[
            pltpu.VMEM((B_PER_W,), jnp.int32),
            pltpu.VMEM((B_PER_W,), jnp.int32),
            pltpu.VMEM((CH * 8, EMB), jnp.float32),
            pltpu.VMEM((CH * 8, EMB), jnp.float32),
            pltpu.VMEM((EMB, B_PER_W), jnp.float32),
            pltpu.VMEM((EMB, B_PER_W), jnp.float32),
            pltpu.SemaphoreType.DMA,
            pltpu.SemaphoreType.DMA,
        ],
        compiler_params=pltpu.CompilerParams(needs_layout_passes=False),
    )
    def gather_k(uid_hbm, mid_hbm, uemb_hbm, memb_hbm, uout_hbm, mout_hbm,
                 uid_v, mid_v, ublk_v, mblk_v, ucols_v, mcols_v, usem, msem):
        wid = lax.axis_index("s") * NC + lax.axis_index("c")
        base = pl.multiple_of(wid * B_PER_W, B_PER_W)
        pltpu.sync_copy(uid_hbm.at[pl.ds(base, B_PER_W)], uid_v)
        pltpu.sync_copy(mid_hbm.at[pl.ds(base, B_PER_W)], mid_v)
        iota = lax.iota(jnp.int32, 16)

        def chunk_body(c, _):
            cb = c * CH
            uvecs = []
            mvecs = []
            for g in range(CH // 16):
                gidx = jnp.full((16,), cb + g * 16, jnp.int32) + iota
                uvecs.append(plsc.load_gather(uid_v, [gidx]))
                mvecs.append(plsc.load_gather(mid_v, [gidx]))

            for g in range(CH // 16):
                for k in range(16):
                    i = g * 16 + k
                    ui = uvecs[g][k]
                    mi = mvecs[g][k]
                    ur0 = pl.multiple_of((ui >> 3) * 8, 8)
                    mr0 = pl.multiple_of((mi >> 3) * 8, 8)
                    pltpu.make_async_copy(
                        uemb_hbm.at[pl.ds(ur0, 8), :],
                        ublk_v.at[pl.ds(i * 8, 8), :], usem).start()
                    pltpu.make_async_copy(
                        memb_hbm.at[pl.ds(mr0, 8), :],
                        mblk_v.at[pl.ds(i * 8, 8), :], msem).start()

            pltpu.make_async_copy(
                uemb_hbm.at[pl.ds(0, CH * 8), :], ublk_v, usem).wait()
            pltpu.make_async_copy(
                memb_hbm.at[pl.ds(0, CH * 8), :], mblk_v, msem).wait()

            for g in range(CH // 16):
                for k in range(16):
                    i = g * 16 + k
                    ui = uvecs[g][k]
                    mi = mvecs[g][k]
                    urow = jnp.full((16,), i * 8 + (ui & 7), jnp.int32)
                    mrow = jnp.full((16,), i * 8 + (mi & 7), jnp.int32)
                    col = jnp.full((16,), cb + i, jnp.int32)
                    for half in range(2):
                        cols = iota + (half * 16)
                        uvals = plsc.load_gather(ublk_v, [urow, cols])
                        mvals = plsc.load_gather(mblk_v, [mrow, cols])
                        plsc.store_scatter(ucols_v, [cols, col], uvals)
                        plsc.store_scatter(mcols_v, [cols, col], mvals)
            return ()

        lax.fori_loop(0, N_CHUNKS, chunk_body, ())
        pltpu.sync_copy(ucols_v, uout_hbm.at[:, pl.ds(base, B_PER_W)])
        pltpu.sync_copy(mcols_v, mout_hbm.at[:, pl.ds(base, B_PER_W)])

    return gather_k


_gather = _make_gather()

_BLK = 2048


def _mlp_body(ut_ref, mt_ref, w1u_ref, w1m_ref, b1_ref, w2_ref, b2_ref,
              o_ref):
    dn = (((0,), (0,)), ((), ()))
    h = (lax.dot_general(ut_ref[...], w1u_ref[...], dn,
                         preferred_element_type=jnp.float32)
         + lax.dot_general(mt_ref[...], w1m_ref[...], dn,
                           preferred_element_type=jnp.float32)
         + b1_ref[...])
    h = jnp.maximum(h, 0.0)
    o_ref[...] = (jnp.dot(h, w2_ref[...], preferred_element_type=jnp.float32)
                  + b2_ref[...])


def _mlp(ut, mt, w1u, w1m, b1, w2, b2):
    return pl.pallas_call(
        _mlp_body,
        grid=(B // _BLK,),
        in_specs=[
            pl.BlockSpec((EMB, _BLK), lambda i: (0, i)),
            pl.BlockSpec((EMB, _BLK), lambda i: (0, i)),
            pl.BlockSpec((EMB, HIDDEN), lambda i: (0, 0)),
            pl.BlockSpec((EMB, HIDDEN), lambda i: (0, 0)),
            pl.BlockSpec((1, HIDDEN), lambda i: (0, 0)),
            pl.BlockSpec((HIDDEN, 1), lambda i: (0, 0)),
            pl.BlockSpec((1, 1), lambda i: (0, 0)),
        ],
        out_specs=pl.BlockSpec((_BLK, 1), lambda i: (i, 0)),
        out_shape=jax.ShapeDtypeStruct((B, 1), jnp.float32),
    )(ut, mt, w1u, w1m, b1, w2, b2)


def kernel(user_ids, movie_ids, user_emb, movie_emb, W1, b1, W2, b2):
    uid = user_ids.astype(jnp.int32)
    mid = movie_ids.astype(jnp.int32)
    ut, mt = _gather(uid, mid, user_emb, movie_emb)
    y = _mlp(ut, mt, W1[:EMB], W1[EMB:], b1.reshape(1, HIDDEN), W2,
             b2.reshape(1, 1))
    return y.reshape(-1)


# speedup vs baseline: 2.2773x; 1.6498x over previous
import functools

import jax
import jax.numpy as jnp
from jax import lax
from jax.experimental import pallas as pl
from jax.experimental.pallas import tpu as pltpu
from jax.experimental.pallas import tpu_sc as plsc

EMB = 32
HIDDEN = 128
B = 16384

NC = 2
NS = 16
NW = NC * NS
B_PER_W = B // NW
UCH = 8
MCH = 32


def _make_gather():
    mesh = plsc.VectorSubcoreMesh(core_axis_name="c", subcore_axis_name="s")

    @functools.partial(
        pl.kernel,
        out_type=[
            jax.ShapeDtypeStruct((EMB, B), jnp.float32),
            jax.ShapeDtypeStruct((EMB, B), jnp.float32),
        ],
        mesh=mesh,
        scratch_types=[
            pltpu.VMEM((B_PER_W,), jnp.int32),
            pltpu.VMEM((B_PER_W,), jnp.int32),
            pltpu.VMEM((UCH * EMB, 128), jnp.float32),
            pltpu.VMEM((MCH * 8, EMB), jnp.float32),
            pltpu.VMEM((EMB, B_PER_W), jnp.float32),
            pltpu.VMEM((EMB, B_PER_W), jnp.float32),
            pltpu.SemaphoreType.DMA,
            pltpu.SemaphoreType.DMA,
        ],
        compiler_params=pltpu.CompilerParams(needs_layout_passes=False),
    )
    def gather_k(uid_hbm, mid_hbm, ut_hbm, memb_hbm, uout_hbm, mout_hbm,
                 uid_v, mid_v, ublk_v, mblk_v, ucols_v, mcols_v, usem, msem):
        wid = lax.axis_index("s") * NC + lax.axis_index("c")
        base = pl.multiple_of(wid * B_PER_W, B_PER_W)
        pltpu.sync_copy(uid_hbm.at[pl.ds(base, B_PER_W)], uid_v)
        pltpu.sync_copy(mid_hbm.at[pl.ds(base, B_PER_W)], mid_v)
        iota = lax.iota(jnp.int32, 16)

        def uchunk_body(c, _):
            cb = c * UCH
            gidx = jnp.full((16,), cb, jnp.int32) + iota
            uvec = plsc.load_gather(uid_v, [gidx])
            for l in range(UCH):
                ui = uvec[l]
                col0 = pl.multiple_of((ui >> 7) * 128, 128)
                pltpu.make_async_copy(
                    ut_hbm.at[:, pl.ds(col0, 128)],
                    ublk_v.at[pl.ds(l * EMB, EMB), :], usem).start()
            for l in range(UCH):
                pltpu.make_async_copy(
                    ut_hbm.at[:, pl.ds(0, 128)],
                    ublk_v.at[pl.ds(l * EMB, EMB), :], usem).wait()
            for l in range(UCH):
                ui = uvec[l]
                lane = jnp.full((16,), ui & 127, jnp.int32)
                col = jnp.full((16,), cb + l, jnp.int32)
                for half in range(2):
                    rows = jnp.full((16,), l * EMB + half * 16, jnp.int32) + iota
                    vals = plsc.load_gather(ublk_v, [rows, lane])
                    plsc.store_scatter(
                        ucols_v, [iota + half * 16, col], vals)
            return ()

        lax.fori_loop(0, B_PER_W // UCH, uchunk_body, ())

        def mchunk_body(c, _):
            cb = c * MCH
            mvecs = []
            for g in range(MCH // 16):
                gidx = jnp.full((16,), cb + g * 16, jnp.int32) + iota
                mvecs.append(plsc.load_gather(mid_v, [gidx]))
            for g in range(MCH // 16):
                for k in range(16):
                    i = g * 16 + k
                    mi = mvecs[g][k]
                    mr0 = pl.multiple_of((mi >> 3) * 8, 8)
                    pltpu.make_async_copy(
                        memb_hbm.at[pl.ds(mr0, 8), :],
                        mblk_v.at[pl.ds(i * 8, 8), :], msem).start()
            pltpu.make_async_copy(
                memb_hbm.at[pl.ds(0, MCH * 8), :], mblk_v, msem).wait()
            for g in range(MCH // 16):
                for k in range(16):
                    i = g * 16 + k
                    mi = mvecs[g][k]
                    mrow = jnp.full((16,), i * 8 + (mi & 7), jnp.int32)
                    col = jnp.full((16,), cb + i, jnp.int32)
                    for half in range(2):
                        cols = iota + (half * 16)
                        vals = plsc.load_gather(mblk_v, [mrow, cols])
                        plsc.store_scatter(mcols_v, [cols, col], vals)
            return ()

        lax.fori_loop(0, B_PER_W // MCH, mchunk_body, ())

        pltpu.sync_copy(ucols_v, uout_hbm.at[:, pl.ds(base, B_PER_W)])
        pltpu.sync_copy(mcols_v, mout_hbm.at[:, pl.ds(base, B_PER_W)])

    return gather_k


_gather = _make_gather()

_BLK = 2048


def _mlp_body(ut_ref, mt_ref, w1u_ref, w1m_ref, b1_ref, w2_ref, b2_ref,
              o_ref):
    dn = (((0,), (0,)), ((), ()))
    h = (lax.dot_general(ut_ref[...], w1u_ref[...], dn,
                         preferred_element_type=jnp.float32)
         + lax.dot_general(mt_ref[...], w1m_ref[...], dn,
                           preferred_element_type=jnp.float32)
         + b1_ref[...])
    h = jnp.maximum(h, 0.0)
    o_ref[...] = (jnp.dot(h, w2_ref[...], preferred_element_type=jnp.float32)
                  + b2_ref[...])


def _mlp(ut, mt, w1u, w1m, b1, w2, b2):
    return pl.pallas_call(
        _mlp_body,
        grid=(B // _BLK,),
        in_specs=[
            pl.BlockSpec((EMB, _BLK), lambda i: (0, i)),
            pl.BlockSpec((EMB, _BLK), lambda i: (0, i)),
            pl.BlockSpec((EMB, HIDDEN), lambda i: (0, 0)),
            pl.BlockSpec((EMB, HIDDEN), lambda i: (0, 0)),
            pl.BlockSpec((1, HIDDEN), lambda i: (0, 0)),
            pl.BlockSpec((HIDDEN, 1), lambda i: (0, 0)),
            pl.BlockSpec((1, 1), lambda i: (0, 0)),
        ],
        out_specs=pl.BlockSpec((_BLK, 1), lambda i: (i, 0)),
        out_shape=jax.ShapeDtypeStruct((B, 1), jnp.float32),
    )(ut, mt, w1u, w1m, b1, w2, b2)


def kernel(user_ids, movie_ids, user_emb, movie_emb, W1, b1, W2, b2):
    uid = user_ids.astype(jnp.int32)
    mid = movie_ids.astype(jnp.int32)
    ut, mt = _gather(uid, mid, user_emb.T, movie_emb)
    y = _mlp(ut, mt, W1[:EMB], W1[EMB:], b1.reshape(1, HIDDEN), W2,
             b2.reshape(1, 1))
    return y.reshape(-1)


# speedup vs baseline: 2.6903x; 1.1814x over previous
import functools

import jax
import jax.numpy as jnp
from jax import lax
from jax.experimental import pallas as pl
from jax.experimental.pallas import tpu as pltpu
from jax.experimental.pallas import tpu_sc as plsc

EMB = 32
HIDDEN = 128
B = 16384

NC = 2
NS = 16
NW = NC * NS
B_PER_W = B // NW
CH = 4
N_CH = B_PER_W // CH


def _make_gather():
    mesh = plsc.VectorSubcoreMesh(core_axis_name="c", subcore_axis_name="s")

    @functools.partial(
        pl.kernel,
        out_type=[
            jax.ShapeDtypeStruct((EMB, B), jnp.float32),
            jax.ShapeDtypeStruct((EMB, B), jnp.float32),
        ],
        mesh=mesh,
        scratch_types=[
            pltpu.VMEM((B_PER_W,), jnp.int32),
            pltpu.VMEM((B_PER_W,), jnp.int32),
            pltpu.VMEM((CH * EMB, 128), jnp.float32),
            pltpu.VMEM((CH * EMB, 128), jnp.float32),
            pltpu.VMEM((CH * 8, EMB), jnp.float32),
            pltpu.VMEM((CH * 8, EMB), jnp.float32),
            pltpu.VMEM((EMB, B_PER_W), jnp.float32),
            pltpu.VMEM((EMB, B_PER_W), jnp.float32),
            pltpu.SemaphoreType.DMA,
            pltpu.SemaphoreType.DMA,
            pltpu.SemaphoreType.DMA,
            pltpu.SemaphoreType.DMA,
        ],
        compiler_params=pltpu.CompilerParams(needs_layout_passes=False),
    )
    def gather_k(uid_hbm, mid_hbm, ut_hbm, memb_hbm, uout_hbm, mout_hbm,
                 uid_v, mid_v, ub0, ub1, mb0, mb1, ucols_v, mcols_v,
                 us0, us1, ms0, ms1):
        wid = lax.axis_index("s") * NC + lax.axis_index("c")
        base = pl.multiple_of(wid * B_PER_W, B_PER_W)
        pltpu.sync_copy(uid_hbm.at[pl.ds(base, B_PER_W)], uid_v)
        pltpu.sync_copy(mid_hbm.at[pl.ds(base, B_PER_W)], mid_v)
        iota = lax.iota(jnp.int32, 16)

        def fire(c, ub, mb, usem, msem):
            gidx = jnp.full((16,), c * CH, jnp.int32) + iota
            uvec = plsc.load_gather(uid_v, [gidx])
            mvec = plsc.load_gather(mid_v, [gidx])
            for l in range(CH):
                ui = uvec[l]
                mi = mvec[l]
                col0 = pl.multiple_of((ui >> 7) * 128, 128)
                pltpu.make_async_copy(
                    ut_hbm.at[:, pl.ds(col0, 128)],
                    ub.at[pl.ds(l * EMB, EMB), :], usem).start()
                mr0 = pl.multiple_of((mi >> 3) * 8, 8)
                pltpu.make_async_copy(
                    memb_hbm.at[pl.ds(mr0, 8), :],
                    mb.at[pl.ds(l * 8, 8), :], msem).start()
            return uvec, mvec

        def drain_extract(c, ub, mb, usem, msem, uvec, mvec):
            for l in range(CH):
                pltpu.make_async_copy(
                    ut_hbm.at[:, pl.ds(0, 128)],
                    ub.at[pl.ds(l * EMB, EMB), :], usem).wait()
            pltpu.make_async_copy(
                memb_hbm.at[pl.ds(0, CH * 8), :], mb, msem).wait()
            for l in range(CH):
                ui = uvec[l]
                mi = mvec[l]
                lane = jnp.full((16,), ui & 127, jnp.int32)
                mrow = jnp.full((16,), l * 8 + (mi & 7), jnp.int32)
                col = jnp.full((16,), c * CH + l, jnp.int32)
                for half in range(2):
                    out_rows = iota + half * 16
                    rows = jnp.full((16,), l * EMB + half * 16,
                                    jnp.int32) + iota
                    uvals = plsc.load_gather(ub, [rows, lane])
                    plsc.store_scatter(ucols_v, [out_rows, col], uvals)
                    mvals = plsc.load_gather(mb, [mrow, out_rows])
                    plsc.store_scatter(mcols_v, [out_rows, col], mvals)

        uv0, mv0 = fire(0, ub0, mb0, us0, ms0)

        def body(c2, carry):
            uva, mva = carry
            ca = 2 * c2
            uvb, mvb = fire(ca + 1, ub1, mb1, us1, ms1)
            drain_extract(ca, ub0, mb0, us0, ms0, uva, mva)
            uvc, mvc = fire(ca + 2, ub0, mb0, us0, ms0)
            drain_extract(ca + 1, ub1, mb1, us1, ms1, uvb, mvb)
            return (uvc, mvc)

        uvl, mvl = lax.fori_loop(0, N_CH // 2 - 1, body, (uv0, mv0))
        uvz, mvz = fire(N_CH - 1, ub1, mb1, us1, ms1)
        drain_extract(N_CH - 2, ub0, mb0, us0, ms0, uvl, mvl)
        drain_extract(N_CH - 1, ub1, mb1, us1, ms1, uvz, mvz)

        pltpu.sync_copy(ucols_v, uout_hbm.at[:, pl.ds(base, B_PER_W)])
        pltpu.sync_copy(mcols_v, mout_hbm.at[:, pl.ds(base, B_PER_W)])

    return gather_k


_gather = _make_gather()

_BLK = 2048


def _mlp_body(ut_ref, mt_ref, w1u_ref, w1m_ref, b1_ref, w2_ref, b2_ref,
              o_ref):
    dn = (((0,), (0,)), ((), ()))
    h = (lax.dot_general(ut_ref[...], w1u_ref[...], dn,
                         preferred_element_type=jnp.float32)
         + lax.dot_general(mt_ref[...], w1m_ref[...], dn,
                           preferred_element_type=jnp.float32)
         + b1_ref[...])
    h = jnp.maximum(h, 0.0)
    o_ref[...] = (jnp.dot(h, w2_ref[...], preferred_element_type=jnp.float32)
                  + b2_ref[...])


def _mlp(ut, mt, w1u, w1m, b1, w2, b2):
    return pl.pallas_call(
        _mlp_body,
        grid=(B // _BLK,),
        in_specs=[
            pl.BlockSpec((EMB, _BLK), lambda i: (0, i)),
            pl.BlockSpec((EMB, _BLK), lambda i: (0, i)),
            pl.BlockSpec((EMB, HIDDEN), lambda i: (0, 0)),
            pl.BlockSpec((EMB, HIDDEN), lambda i: (0, 0)),
            pl.BlockSpec((1, HIDDEN), lambda i: (0, 0)),
            pl.BlockSpec((HIDDEN, 1), lambda i: (0, 0)),
            pl.BlockSpec((1, 1), lambda i: (0, 0)),
        ],
        out_specs=pl.BlockSpec((_BLK, 1), lambda i: (i, 0)),
        out_shape=jax.ShapeDtypeStruct((B, 1), jnp.float32),
    )(ut, mt, w1u, w1m, b1, w2, b2)


def kernel(user_ids, movie_ids, user_emb, movie_emb, W1, b1, W2, b2):
    uid = user_ids.astype(jnp.int32)
    mid = movie_ids.astype(jnp.int32)
    ut, mt = _gather(uid, mid, user_emb.T, movie_emb)
    y = _mlp(ut, mt, W1[:EMB], W1[EMB:], b1.reshape(1, HIDDEN), W2,
             b2.reshape(1, 1))
    return y.reshape(-1)


# speedup vs baseline: 2.8271x; 1.0508x over previous
import functools

import jax
import jax.numpy as jnp
from jax import lax
from jax.experimental import pallas as pl
from jax.experimental.pallas import tpu as pltpu
from jax.experimental.pallas import tpu_sc as plsc

EMB = 32
HIDDEN = 128
B = 16384

NC = 2
NS = 16
NW = NC * NS
B_PER_W = B // NW
CH = 8
N_CH = B_PER_W // CH


def _make_gather():
    mesh = plsc.VectorSubcoreMesh(core_axis_name="c", subcore_axis_name="s")

    @functools.partial(
        pl.kernel,
        out_type=[
            jax.ShapeDtypeStruct((EMB, B), jnp.float32),
            jax.ShapeDtypeStruct((EMB, B), jnp.float32),
        ],
        mesh=mesh,
        scratch_types=[
            pltpu.VMEM((B_PER_W,), jnp.int32),
            pltpu.VMEM((B_PER_W,), jnp.int32),
            pltpu.VMEM((CH * EMB, 128), jnp.float32),
            pltpu.VMEM((CH * EMB, 128), jnp.float32),
            pltpu.VMEM((CH * 8, EMB), jnp.float32),
            pltpu.VMEM((CH * 8, EMB), jnp.float32),
            pltpu.VMEM((EMB, B_PER_W), jnp.float32),
            pltpu.VMEM((EMB, B_PER_W), jnp.float32),
            pltpu.SemaphoreType.DMA,
            pltpu.SemaphoreType.DMA,
            pltpu.SemaphoreType.DMA,
            pltpu.SemaphoreType.DMA,
        ],
        compiler_params=pltpu.CompilerParams(needs_layout_passes=False),
    )
    def gather_k(uid_hbm, mid_hbm, ut_hbm, memb_hbm, uout_hbm, mout_hbm,
                 uid_v, mid_v, ub0, ub1, mb0, mb1, ucols_v, mcols_v,
                 us0, us1, ms0, ms1):
        wid = lax.axis_index("s") * NC + lax.axis_index("c")
        base = pl.multiple_of(wid * B_PER_W, B_PER_W)
        pltpu.sync_copy(uid_hbm.at[pl.ds(base, B_PER_W)], uid_v)
        pltpu.sync_copy(mid_hbm.at[pl.ds(base, B_PER_W)], mid_v)
        iota = lax.iota(jnp.int32, 16)

        def fire(c, ub, mb, usem, msem):
            gidx = jnp.full((16,), c * CH, jnp.int32) + iota
            uvec = plsc.load_gather(uid_v, [gidx])
            mvec = plsc.load_gather(mid_v, [gidx])
            for l in range(CH):
                ui = uvec[l]
                mi = mvec[l]
                col0 = pl.multiple_of((ui >> 7) * 128, 128)
                pltpu.make_async_copy(
                    ut_hbm.at[:, pl.ds(col0, 128)],
                    ub.at[pl.ds(l * EMB, EMB), :], usem).start()
                mr0 = pl.multiple_of((mi >> 3) * 8, 8)
                pltpu.make_async_copy(
                    memb_hbm.at[pl.ds(mr0, 8), :],
                    mb.at[pl.ds(l * 8, 8), :], msem).start()
            return uvec, mvec

        def drain_extract(c, ub, mb, usem, msem, uvec, mvec):
            for l in range(CH):
                pltpu.make_async_copy(
                    ut_hbm.at[:, pl.ds(0, 128)],
                    ub.at[pl.ds(l * EMB, EMB), :], usem).wait()
            pltpu.make_async_copy(
                memb_hbm.at[pl.ds(0, CH * 8), :], mb, msem).wait()
            for l in range(CH):
                ui = uvec[l]
                mi = mvec[l]
                lane = jnp.full((16,), ui & 127, jnp.int32)
                mrow = jnp.full((16,), l * 8 + (mi & 7), jnp.int32)
                col = jnp.full((16,), c * CH + l, jnp.int32)
                for half in range(2):
                    out_rows = iota + half * 16
                    rows = jnp.full((16,), l * EMB + half * 16,
                                    jnp.int32) + iota
                    uvals = plsc.load_gather(ub, [rows, lane])
                    plsc.store_scatter(ucols_v, [out_rows, col], uvals)
                    mvals = plsc.load_gather(mb, [mrow, out_rows])
                    plsc.store_scatter(mcols_v, [out_rows, col], mvals)

        uv0, mv0 = fire(0, ub0, mb0, us0, ms0)

        def body(c2, carry):
            uva, mva = carry
            ca = 2 * c2
            uvb, mvb = fire(ca + 1, ub1, mb1, us1, ms1)
            drain_extract(ca, ub0, mb0, us0, ms0, uva, mva)
            uvc, mvc = fire(ca + 2, ub0, mb0, us0, ms0)
            drain_extract(ca + 1, ub1, mb1, us1, ms1, uvb, mvb)
            return (uvc, mvc)

        uvl, mvl = lax.fori_loop(0, N_CH // 2 - 1, body, (uv0, mv0))
        uvz, mvz = fire(N_CH - 1, ub1, mb1, us1, ms1)
        drain_extract(N_CH - 2, ub0, mb0, us0, ms0, uvl, mvl)
        drain_extract(N_CH - 1, ub1, mb1, us1, ms1, uvz, mvz)

        pltpu.sync_copy(ucols_v, uout_hbm.at[:, pl.ds(base, B_PER_W)])
        pltpu.sync_copy(mcols_v, mout_hbm.at[:, pl.ds(base, B_PER_W)])

    return gather_k


_gather = _make_gather()

_BLK = 2048


def _mlp_body(ut_ref, mt_ref, w1u_ref, w1m_ref, b1_ref, w2_ref, b2_ref,
              o_ref):
    dn = (((0,), (0,)), ((), ()))
    h = (lax.dot_general(ut_ref[...], w1u_ref[...], dn,
                         preferred_element_type=jnp.float32)
         + lax.dot_general(mt_ref[...], w1m_ref[...], dn,
                           preferred_element_type=jnp.float32)
         + b1_ref[...])
    h = jnp.maximum(h, 0.0)
    o_ref[...] = (jnp.dot(h, w2_ref[...], preferred_element_type=jnp.float32)
                  + b2_ref[...])


def _mlp(ut, mt, w1u, w1m, b1, w2, b2):
    return pl.pallas_call(
        _mlp_body,
        grid=(B // _BLK,),
        in_specs=[
            pl.BlockSpec((EMB, _BLK), lambda i: (0, i)),
            pl.BlockSpec((EMB, _BLK), lambda i: (0, i)),
            pl.BlockSpec((EMB, HIDDEN), lambda i: (0, 0)),
            pl.BlockSpec((EMB, HIDDEN), lambda i: (0, 0)),
            pl.BlockSpec((1, HIDDEN), lambda i: (0, 0)),
            pl.BlockSpec((HIDDEN, 1), lambda i: (0, 0)),
            pl.BlockSpec((1, 1), lambda i: (0, 0)),
        ],
        out_specs=pl.BlockSpec((_BLK, 1), lambda i: (i, 0)),
        out_shape=jax.ShapeDtypeStruct((B, 1), jnp.float32),
    )(ut, mt, w1u, w1m, b1, w2, b2)


def kernel(user_ids, movie_ids, user_emb, movie_emb, W1, b1, W2, b2):
    uid = user_ids.astype(jnp.int32)
    mid = movie_ids.astype(jnp.int32)
    ut, mt = _gather(uid, mid, user_emb.T, movie_emb)
    y = _mlp(ut, mt, W1[:EMB], W1[EMB:], b1.reshape(1, HIDDEN), W2,
             b2.reshape(1, 1))
    return y.reshape(-1)


# speedup vs baseline: 2.8431x; 1.0056x over previous
import functools

import jax
import jax.numpy as jnp
from jax import lax
from jax.experimental import pallas as pl
from jax.experimental.pallas import tpu as pltpu
from jax.experimental.pallas import tpu_sc as plsc

EMB = 32
HIDDEN = 128
B = 16384

NC = 2
NS = 16
NW = NC * NS
B_PER_W = B // NW
CH = 8
N_CH = B_PER_W // CH


def _make_gather():
    mesh = plsc.VectorSubcoreMesh(core_axis_name="c", subcore_axis_name="s")

    @functools.partial(
        pl.kernel,
        out_type=[
            jax.ShapeDtypeStruct((EMB, B), jnp.float32),
            jax.ShapeDtypeStruct((EMB, B), jnp.float32),
        ],
        mesh=mesh,
        scratch_types=[
            pltpu.VMEM((B_PER_W,), jnp.int32),
            pltpu.VMEM((B_PER_W,), jnp.int32),
            pltpu.VMEM((CH * EMB, 128), jnp.float32),
            pltpu.VMEM((CH * EMB, 128), jnp.float32),
            pltpu.VMEM((CH * 8, EMB), jnp.float32),
            pltpu.VMEM((CH * 8, EMB), jnp.float32),
            pltpu.VMEM((EMB, B_PER_W), jnp.float32),
            pltpu.VMEM((EMB, B_PER_W), jnp.float32),
            pltpu.SemaphoreType.DMA,
            pltpu.SemaphoreType.DMA,
            pltpu.SemaphoreType.DMA,
            pltpu.SemaphoreType.DMA,
        ],
        compiler_params=pltpu.CompilerParams(needs_layout_passes=False),
    )
    def gather_k(uid_hbm, mid_hbm, ut_hbm, memb_hbm, uout_hbm, mout_hbm,
                 uid_v, mid_v, ub0, ub1, mb0, mb1, ucols_v, mcols_v,
                 us0, us1, ms0, ms1):
        wid = lax.axis_index("s") * NC + lax.axis_index("c")
        base = pl.multiple_of(wid * B_PER_W, B_PER_W)
        pltpu.sync_copy(uid_hbm.at[pl.ds(base, B_PER_W)], uid_v)
        pltpu.sync_copy(mid_hbm.at[pl.ds(base, B_PER_W)], mid_v)
        iota = lax.iota(jnp.int32, 16)

        def fire(c, ub, mb, usem, msem):
            gidx = jnp.full((16,), c * CH, jnp.int32) + iota
            uvec = plsc.load_gather(uid_v, [gidx])
            mvec = plsc.load_gather(mid_v, [gidx])
            for l in range(CH):
                ui = uvec[l]
                mi = mvec[l]
                col0 = pl.multiple_of((ui >> 7) * 128, 128)
                pltpu.make_async_copy(
                    ut_hbm.at[:, pl.ds(col0, 128)],
                    ub.at[pl.ds(l * EMB, EMB), :], usem).start()
                mr0 = pl.multiple_of((mi >> 3) * 8, 8)
                pltpu.make_async_copy(
                    memb_hbm.at[pl.ds(mr0, 8), :],
                    mb.at[pl.ds(l * 8, 8), :], msem).start()
            return uvec, mvec

        def drain_extract(c, ub, mb, usem, msem, uvec, mvec):
            for l in range(CH):
                pltpu.make_async_copy(
                    ut_hbm.at[:, pl.ds(0, 128)],
                    ub.at[pl.ds(l * EMB, EMB), :], usem).wait()
            pltpu.make_async_copy(
                memb_hbm.at[pl.ds(0, CH * 8), :], mb, msem).wait()
            for l in range(CH):
                ui = uvec[l]
                mi = mvec[l]
                lane = jnp.full((16,), ui & 127, jnp.int32)
                mrow = jnp.full((16,), l * 8 + (mi & 7), jnp.int32)
                col = jnp.full((16,), c * CH + l, jnp.int32)
                for half in range(2):
                    out_rows = iota + half * 16
                    rows = jnp.full((16,), l * EMB + half * 16,
                                    jnp.int32) + iota
                    uvals = plsc.load_gather(ub, [rows, lane])
                    plsc.store_scatter(ucols_v, [out_rows, col], uvals)
                    mvals = plsc.load_gather(mb, [mrow, out_rows])
                    plsc.store_scatter(mcols_v, [out_rows, col], mvals)

        uv0, mv0 = fire(0, ub0, mb0, us0, ms0)

        def body(c2, carry):
            uva, mva = carry
            ca = 2 * c2
            uvb, mvb = fire(ca + 1, ub1, mb1, us1, ms1)
            drain_extract(ca, ub0, mb0, us0, ms0, uva, mva)
            uvc, mvc = fire(ca + 2, ub0, mb0, us0, ms0)
            drain_extract(ca + 1, ub1, mb1, us1, ms1, uvb, mvb)
            return (uvc, mvc)

        uvl, mvl = lax.fori_loop(0, N_CH // 2 - 1, body, (uv0, mv0))
        uvz, mvz = fire(N_CH - 1, ub1, mb1, us1, ms1)
        drain_extract(N_CH - 2, ub0, mb0, us0, ms0, uvl, mvl)
        drain_extract(N_CH - 1, ub1, mb1, us1, ms1, uvz, mvz)

        pltpu.sync_copy(ucols_v, uout_hbm.at[:, pl.ds(base, B_PER_W)])
        pltpu.sync_copy(mcols_v, mout_hbm.at[:, pl.ds(base, B_PER_W)])

    return gather_k


_gather = _make_gather()

_BLK = 2048


def _mlp_body(ut_ref, mt_ref, w1u_ref, w1m_ref, b1_ref, w2_ref, b2_ref,
              o_ref):
    dn = (((0,), (0,)), ((), ()))
    h = (lax.dot_general(ut_ref[...], w1u_ref[...], dn,
                         preferred_element_type=jnp.float32)
         + lax.dot_general(mt_ref[...], w1m_ref[...], dn,
                           preferred_element_type=jnp.float32)
         + b1_ref[...])
    h = jnp.maximum(h, 0.0)
    o_ref[...] = jnp.sum(h * w2_ref[...], axis=1) + b2_ref[0, 0]


def _mlp(ut, mt, w1u, w1m, b1, w2, b2):
    return pl.pallas_call(
        _mlp_body,
        grid=(B // _BLK,),
        in_specs=[
            pl.BlockSpec((EMB, _BLK), lambda i: (0, i)),
            pl.BlockSpec((EMB, _BLK), lambda i: (0, i)),
            pl.BlockSpec((EMB, HIDDEN), lambda i: (0, 0)),
            pl.BlockSpec((EMB, HIDDEN), lambda i: (0, 0)),
            pl.BlockSpec((1, HIDDEN), lambda i: (0, 0)),
            pl.BlockSpec((1, HIDDEN), lambda i: (0, 0)),
            pl.BlockSpec((1, 1), lambda i: (0, 0)),
        ],
        out_specs=pl.BlockSpec((_BLK,), lambda i: (i,)),
        out_shape=jax.ShapeDtypeStruct((B,), jnp.float32),
    )(ut, mt, w1u, w1m, b1, w2, b2)


def kernel(user_ids, movie_ids, user_emb, movie_emb, W1, b1, W2, b2):
    uid = user_ids.astype(jnp.int32)
    mid = movie_ids.astype(jnp.int32)
    ut, mt = _gather(uid, mid, user_emb.T, movie_emb)
    y = _mlp(ut, mt, W1[:EMB], W1[EMB:], b1.reshape(1, HIDDEN),
             W2.reshape(1, HIDDEN), b2.reshape(1, 1))
    return y


# speedup vs baseline: 2.8560x; 1.0045x over previous
import functools

import jax
import jax.numpy as jnp
from jax import lax
from jax.experimental import pallas as pl
from jax.experimental.pallas import tpu as pltpu
from jax.experimental.pallas import tpu_sc as plsc

EMB = 32
HIDDEN = 128
B = 16384

NC = 2
NS = 16
NW = NC * NS
B_PER_W = B // NW
CH = 8
N_CH = B_PER_W // CH


def _make_gather():
    mesh = plsc.VectorSubcoreMesh(core_axis_name="c", subcore_axis_name="s")

    @functools.partial(
        pl.kernel,
        out_type=[
            jax.ShapeDtypeStruct((EMB, B), jnp.float32),
            jax.ShapeDtypeStruct((EMB, B), jnp.float32),
        ],
        mesh=mesh,
        scratch_types=[
            pltpu.VMEM((B_PER_W,), jnp.int32),
            pltpu.VMEM((B_PER_W,), jnp.int32),
            pltpu.VMEM((CH * EMB, 128), jnp.float32),
            pltpu.VMEM((CH * EMB, 128), jnp.float32),
            pltpu.VMEM((CH * 8, EMB), jnp.float32),
            pltpu.VMEM((CH * 8, EMB), jnp.float32),
            pltpu.VMEM((EMB, B_PER_W), jnp.float32),
            pltpu.VMEM((EMB, B_PER_W), jnp.float32),
            pltpu.SemaphoreType.DMA,
            pltpu.SemaphoreType.DMA,
            pltpu.SemaphoreType.DMA,
            pltpu.SemaphoreType.DMA,
        ],
        compiler_params=pltpu.CompilerParams(needs_layout_passes=False),
    )
    def gather_k(uid_hbm, mid_hbm, ut_hbm, memb_hbm, uout_hbm, mout_hbm,
                 uid_v, mid_v, ub0, ub1, mb0, mb1, ucols_v, mcols_v,
                 us0, us1, ms0, ms1):
        wid = lax.axis_index("s") * NC + lax.axis_index("c")
        base = pl.multiple_of(wid * B_PER_W, B_PER_W)
        pltpu.sync_copy(uid_hbm.at[pl.ds(base, B_PER_W)], uid_v)
        pltpu.sync_copy(mid_hbm.at[pl.ds(base, B_PER_W)], mid_v)
        iota = lax.iota(jnp.int32, 16)

        def fire(c, ub, mb, usem, msem):
            gidx = jnp.full((16,), c * CH, jnp.int32) + iota
            uvec = plsc.load_gather(uid_v, [gidx])
            mvec = plsc.load_gather(mid_v, [gidx])
            for l in range(CH):
                ui = uvec[l]
                mi = mvec[l]
                col0 = pl.multiple_of((ui >> 7) * 128, 128)
                pltpu.make_async_copy(
                    ut_hbm.at[:, pl.ds(col0, 128)],
                    ub.at[pl.ds(l * EMB, EMB), :], usem).start()
                mr0 = pl.multiple_of((mi >> 3) * 8, 8)
                pltpu.make_async_copy(
                    memb_hbm.at[pl.ds(mr0, 8), :],
                    mb.at[pl.ds(l * 8, 8), :], msem).start()
            return uvec, mvec

        def drain_extract(c, ub, mb, usem, msem, uvec, mvec):
            for l in range(CH):
                pltpu.make_async_copy(
                    ut_hbm.at[:, pl.ds(0, 128)],
                    ub.at[pl.ds(l * EMB, EMB), :], usem).wait()
            pltpu.make_async_copy(
                memb_hbm.at[pl.ds(0, CH * 8), :], mb, msem).wait()
            for l in range(CH):
                ui = uvec[l]
                mi = mvec[l]
                lane = jnp.full((16,), ui & 127, jnp.int32)
                mrow = jnp.full((16,), l * 8 + (mi & 7), jnp.int32)
                col = jnp.full((16,), c * CH + l, jnp.int32)
                for half in range(2):
                    out_rows = iota + half * 16
                    rows = jnp.full((16,), l * EMB + half * 16,
                                    jnp.int32) + iota
                    uvals = plsc.load_gather(ub, [rows, lane])
                    plsc.store_scatter(ucols_v, [out_rows, col], uvals)
                    mvals = plsc.load_gather(mb, [mrow, out_rows])
                    plsc.store_scatter(mcols_v, [out_rows, col], mvals)

        uv0, mv0 = fire(0, ub0, mb0, us0, ms0)

        def body(c2, carry):
            uva, mva = carry
            ca = 2 * c2
            uvb, mvb = fire(ca + 1, ub1, mb1, us1, ms1)
            drain_extract(ca, ub0, mb0, us0, ms0, uva, mva)
            uvc, mvc = fire(ca + 2, ub0, mb0, us0, ms0)
            drain_extract(ca + 1, ub1, mb1, us1, ms1, uvb, mvb)
            return (uvc, mvc)

        uvl, mvl = lax.fori_loop(0, N_CH // 2 - 1, body, (uv0, mv0))
        uvz, mvz = fire(N_CH - 1, ub1, mb1, us1, ms1)
        drain_extract(N_CH - 2, ub0, mb0, us0, ms0, uvl, mvl)
        drain_extract(N_CH - 1, ub1, mb1, us1, ms1, uvz, mvz)

        pltpu.sync_copy(ucols_v, uout_hbm.at[:, pl.ds(base, B_PER_W)])
        pltpu.sync_copy(mcols_v, mout_hbm.at[:, pl.ds(base, B_PER_W)])

    return gather_k


_gather = _make_gather()

_BLK = 4096


def _mlp_body(ut_ref, mt_ref, w1u_ref, w1m_ref, b1_ref, w2_ref, b2_ref,
              o_ref):
    dn = (((0,), (0,)), ((), ()))
    h = (lax.dot_general(ut_ref[...], w1u_ref[...], dn,
                         preferred_element_type=jnp.float32)
         + lax.dot_general(mt_ref[...], w1m_ref[...], dn,
                           preferred_element_type=jnp.float32)
         + b1_ref[...])
    h = jnp.maximum(h, 0.0)
    o_ref[...] = jnp.sum(h * w2_ref[...], axis=1) + b2_ref[0, 0]


def _mlp(ut, mt, w1u, w1m, b1, w2, b2):
    return pl.pallas_call(
        _mlp_body,
        grid=(B // _BLK,),
        in_specs=[
            pl.BlockSpec((EMB, _BLK), lambda i: (0, i)),
            pl.BlockSpec((EMB, _BLK), lambda i: (0, i)),
            pl.BlockSpec((EMB, HIDDEN), lambda i: (0, 0)),
            pl.BlockSpec((EMB, HIDDEN), lambda i: (0, 0)),
            pl.BlockSpec((1, HIDDEN), lambda i: (0, 0)),
            pl.BlockSpec((1, HIDDEN), lambda i: (0, 0)),
            pl.BlockSpec((1, 1), lambda i: (0, 0)),
        ],
        out_specs=pl.BlockSpec((_BLK,), lambda i: (i,)),
        out_shape=jax.ShapeDtypeStruct((B,), jnp.float32),
    )(ut, mt, w1u, w1m, b1, w2, b2)


def kernel(user_ids, movie_ids, user_emb, movie_emb, W1, b1, W2, b2):
    uid = user_ids.astype(jnp.int32)
    mid = movie_ids.astype(jnp.int32)
    ut, mt = _gather(uid, mid, user_emb.T, movie_emb)
    y = _mlp(ut, mt, W1[:EMB], W1[EMB:], b1.reshape(1, HIDDEN),
             W2.reshape(1, HIDDEN), b2.reshape(1, 1))
    return y
